# Initial kernel scaffold; baseline (speedup 1.0000x reference)
import jax
import jax.numpy as jnp
from jax.experimental import pallas as pl


def kernel(idx, table, targets):
    raise NotImplementedError("write your pallas kernel here")


# speedup vs baseline: 1.1542x; 1.1542x over previous
import jax
import jax.numpy as jnp
from jax.experimental import pallas as pl
from jax.experimental.pallas import tpu as pltpu

_ROW_TILE = 256
_LANES = 128
_N_ACC = 4


def _stats_kernel(table_ref, d_ref):
    x = table_ref[...]
    m = jnp.max(x, axis=-1, keepdims=True)
    s = jnp.sum(jnp.exp(x - m), axis=-1, keepdims=True)
    d_ref[...] = (jnp.log(s) + m) - x


def _make_main_kernel(tm, v_shift):
    def _main_kernel(flat_ref, table_ref, d_ref, out_ref, part_ref):
        base = pl.program_id(0) * tm
        accs = [jnp.zeros((1, _LANES), jnp.float32) for _ in range(_N_ACC)]
        for m in range(tm):
            f = flat_ref[base + m]
            i = f >> v_shift
            c = f >> 7
            l = f & (_LANES - 1)
            out_ref[m] = table_ref[i]
            chunk = d_ref[c]
            accs[m % _N_ACC] = accs[m % _N_ACC] + pltpu.roll(chunk, -l, axis=1)
        acc = (accs[0] + accs[1]) + (accs[2] + accs[3])
        part_ref[...] = acc.reshape(1, 1, _LANES)
    return _main_kernel


def kernel(idx, table, targets):
    B, T = idx.shape
    V = table.shape[0]
    N = B * T
    v_shift = (V - 1).bit_length()
    chunks_per_row = V // _LANES
    tm = min(_ROW_TILE, N)
    n_tiles = N // tm

    flat = (idx.reshape(N).astype(jnp.int32) * V
            + targets.reshape(N).astype(jnp.int32))

    vb = min(256, V)
    d = pl.pallas_call(
        _stats_kernel,
        out_shape=jax.ShapeDtypeStruct((V, V), jnp.float32),
        grid=(V // vb,),
        in_specs=[pl.BlockSpec((vb, V), lambda i: (i, 0))],
        out_specs=pl.BlockSpec((vb, V), lambda i: (i, 0)),
        compiler_params=pltpu.CompilerParams(
            dimension_semantics=("parallel",),
            vmem_limit_bytes=32 * 1024 * 1024,
        ),
    )(table)

    table3 = table.reshape(V, 1, V)
    d_chunks = d.reshape(V * chunks_per_row, 1, _LANES)

    logits3, partials = pl.pallas_call(
        _make_main_kernel(tm, v_shift),
        out_shape=(
            jax.ShapeDtypeStruct((N, 1, V), jnp.float32),
            jax.ShapeDtypeStruct((n_tiles, 1, _LANES), jnp.float32),
        ),
        grid_spec=pltpu.PrefetchScalarGridSpec(
            num_scalar_prefetch=1,
            grid=(n_tiles,),
            in_specs=[
                pl.BlockSpec((V, 1, V), lambda i, flat_ref: (0, 0, 0)),
                pl.BlockSpec((V * chunks_per_row, 1, _LANES),
                             lambda i, flat_ref: (0, 0, 0)),
            ],
            out_specs=(
                pl.BlockSpec((tm, 1, V), lambda i, flat_ref: (i, 0, 0)),
                pl.BlockSpec((1, 1, _LANES), lambda i, flat_ref: (i, 0, 0)),
            ),
        ),
        compiler_params=pltpu.CompilerParams(
            dimension_semantics=("parallel",),
            vmem_limit_bytes=56 * 1024 * 1024,
        ),
        cost_estimate=pl.CostEstimate(
            flops=2 * N * V,
            transcendentals=0,
            bytes_accessed=N * V * 4 + 2 * V * V * 4 + N * 4,
        ),
    )(flat, table3, d_chunks)

    logits = logits3.reshape(N, V)
    loss = jnp.sum(partials[:, 0, 0]) * (1.0 / N)
    return logits, loss


# speedup vs baseline: 2.0858x; 1.8071x over previous
import jax
import jax.numpy as jnp
from jax.experimental import pallas as pl
from jax.experimental.pallas import tpu as pltpu

_ROW_TILE = 256
_GROUP = 64
_STRIDE = _GROUP + 1
_LANES = 128
_N_ACC = 4


def _stats_kernel(table_ref, d_ref):
    x = table_ref[...]
    m = jnp.max(x, axis=-1, keepdims=True)
    s = jnp.sum(jnp.exp(x - m), axis=-1, keepdims=True)
    d_ref[...] = (jnp.log(s) + m) - x


def _make_main_kernel(tm, v_shift, chunks):
    def _main_kernel(flat_ref, table_ref, d_ref, out_ref, part_ref,
                     ts_a, ts_b):
        base = pl.program_id(0) * tm
        accs = [jnp.zeros((1, _LANES), jnp.float32) for _ in range(_N_ACC)]
        for g in range(tm // _GROUP):
            ts = ts_a if g % 2 == 0 else ts_b
            for mi in range(_GROUP):
                m = g * _GROUP + mi
                f = flat_ref[base + m]
                i16 = pl.multiple_of((f >> v_shift) * chunks,
                                     8 if chunks % 8 == 0 else chunks)
                slab = table_ref[pl.ds(i16, chunks), :]
                ts[mi:mi + (chunks - 1) * _STRIDE + 1:_STRIDE, :] = slab
                chunk = d_ref[f >> 7]
                accs[m % _N_ACC] = accs[m % _N_ACC] + pltpu.roll(
                    chunk, -(f & (_LANES - 1)), axis=1)
            r0 = g * _GROUP
            for j in range(chunks):
                out_ref[r0:r0 + _GROUP, j * _LANES:(j + 1) * _LANES] = (
                    ts[j * _STRIDE:j * _STRIDE + _GROUP, :])
        acc = (accs[0] + accs[1]) + (accs[2] + accs[3])
        part_ref[...] = acc.reshape(1, 1, _LANES)
    return _main_kernel


def kernel(idx, table, targets):
    B, T = idx.shape
    V = table.shape[0]
    N = B * T
    v_shift = (V - 1).bit_length()
    chunks = V // _LANES
    tm = min(_ROW_TILE, N)
    n_tiles = N // tm

    flat = (idx.reshape(N).astype(jnp.int32) * V
            + targets.reshape(N).astype(jnp.int32))

    vb = min(256, V)
    d = pl.pallas_call(
        _stats_kernel,
        out_shape=jax.ShapeDtypeStruct((V, V), jnp.float32),
        grid=(V // vb,),
        in_specs=[pl.BlockSpec((vb, V), lambda i: (i, 0))],
        out_specs=pl.BlockSpec((vb, V), lambda i: (i, 0)),
        compiler_params=pltpu.CompilerParams(
            dimension_semantics=("parallel",),
            vmem_limit_bytes=32 * 1024 * 1024,
        ),
    )(table)

    table2 = table.reshape(V * chunks, _LANES)
    d_chunks = d.reshape(V * chunks, 1, _LANES)

    scratch = pltpu.VMEM(((chunks - 1) * _STRIDE + _GROUP, _LANES),
                         jnp.float32)
    logits, partials = pl.pallas_call(
        _make_main_kernel(tm, v_shift, chunks),
        out_shape=(
            jax.ShapeDtypeStruct((N, V), jnp.float32),
            jax.ShapeDtypeStruct((n_tiles, 1, _LANES), jnp.float32),
        ),
        grid_spec=pltpu.PrefetchScalarGridSpec(
            num_scalar_prefetch=1,
            grid=(n_tiles,),
            in_specs=[
                pl.BlockSpec((V * chunks, _LANES),
                             lambda i, flat_ref: (0, 0)),
                pl.BlockSpec((V * chunks, 1, _LANES),
                             lambda i, flat_ref: (0, 0, 0)),
            ],
            out_specs=(
                pl.BlockSpec((tm, V), lambda i, flat_ref: (i, 0)),
                pl.BlockSpec((1, 1, _LANES), lambda i, flat_ref: (i, 0, 0)),
            ),
            scratch_shapes=[scratch, scratch],
        ),
        compiler_params=pltpu.CompilerParams(
            dimension_semantics=("parallel",),
            vmem_limit_bytes=56 * 1024 * 1024,
        ),
        cost_estimate=pl.CostEstimate(
            flops=2 * N * V,
            transcendentals=0,
            bytes_accessed=N * V * 4 + 2 * V * V * 4 + N * 4,
        ),
    )(flat, table2, d_chunks)

    loss = jnp.sum(partials[:, 0, 0]) * (1.0 / N)
    return logits, loss
